# baseline (speedup 1.0000x reference)
import functools

import jax
import jax.numpy as jnp
from jax import lax
from jax.experimental import pallas as pl
from jax.experimental.pallas import tpu as pltpu
from jax.experimental.pallas import tpu_sc as plsc

N = 16777216

NC = 2
NS = 16
NW = NC * NS
PER_W = N // NW
CHUNK = 8192
NCH = PER_W // CHUNK
NB = 4
LANES = 16

_mesh = plsc.VectorSubcoreMesh(core_axis_name="c", subcore_axis_name="s")

_scratch = (
    [pltpu.VMEM((CHUNK,), jnp.float32) for _ in range(2 * NB)]
    + [pltpu.SemaphoreType.DMA for _ in range(2 * NB)]
)


@functools.partial(
    pl.kernel,
    out_type=jax.ShapeDtypeStruct((N,), jnp.float32),
    mesh=_mesh,
    scratch_types=_scratch,
)
def _sc_dy(z_hbm, dy_hbm, *bufs):
    xin = bufs[0:NB]
    dyo = bufs[NB:2 * NB]
    in_sem = bufs[2 * NB:3 * NB]
    out_sem = bufs[3 * NB:4 * NB]

    wid = lax.axis_index("s") * NC + lax.axis_index("c")
    base = wid * PER_W

    neg1 = jnp.full((LANES,), -1.0, jnp.float32)

    def start_in(b, ch):
        off = base + ch * CHUNK
        pltpu.async_copy(z_hbm.at[0, pl.ds(off, CHUNK)], xin[b], in_sem[b])

    def wait_in(b):
        pltpu.make_async_copy(z_hbm.at[0, pl.ds(0, CHUNK)], xin[b], in_sem[b]).wait()

    def start_out(b, ch):
        off = base + ch * CHUNK
        pltpu.async_copy(dyo[b], dy_hbm.at[pl.ds(off, CHUNK)], out_sem[b])

    def wait_out(b):
        pltpu.make_async_copy(dyo[b], dy_hbm.at[pl.ds(0, CHUNK)], out_sem[b]).wait()

    def compute(b):
        xv, dyv = xin[b], dyo[b]

        @plsc.parallel_loop(0, CHUNK, step=LANES, unroll=8)
        def _(i):
            s = pl.ds(i, LANES)
            x = xv[s]
            dyv[s] = jnp.where(x >= 2.0, x + 2.0, neg1)

    for b in range(NB):
        start_in(b, jnp.int32(b))

    def group(g, _):
        for b in range(NB):
            ch = g * NB + b
            wait_in(b)

            @pl.when(g > 0)
            def _():
                wait_out(b)

            compute(b)
            start_out(b, ch)

            @pl.when(ch + NB < NCH)
            def _():
                start_in(b, ch + NB)
        return 0

    lax.fori_loop(0, NCH // NB, group, 0)
    for b in range(NB):
        wait_out(b)


TC_BLOCK = 1048576


def _tc_dx_body(z_ref, dx_ref):
    x = z_ref[0, :]
    y = z_ref[1, :]
    dx_ref[...] = jnp.where(x >= 2.0, -y, jnp.where(y >= 0.0, -1.0, 1.0))


def kernel(t, z):
    dy = _sc_dy(z)
    dx = pl.pallas_call(
        _tc_dx_body,
        grid=(N // TC_BLOCK,),
        in_specs=[pl.BlockSpec((2, TC_BLOCK), lambda i: (0, i))],
        out_specs=pl.BlockSpec((TC_BLOCK,), lambda i: (i,)),
        out_shape=jax.ShapeDtypeStruct((N,), jnp.float32),
    )(z)
    return (dx, dy)


# speedup vs baseline: 1.0939x; 1.0939x over previous
import functools

import jax
import jax.numpy as jnp
from jax import lax
from jax.experimental import pallas as pl
from jax.experimental.pallas import tpu as pltpu
from jax.experimental.pallas import tpu_sc as plsc

N = 16777216
NC = 2
NS = 16
NW = NC * NS
PER_W = N // NW
CHUNK = 2048
NCH = PER_W // CHUNK
NB = 8
LANES = 16

_mesh = plsc.VectorSubcoreMesh(core_axis_name="c", subcore_axis_name="s")

_scratch = (
    [pltpu.VMEM((2, CHUNK), jnp.float32) for _ in range(NB)]
    + [pltpu.VMEM((CHUNK,), jnp.float32) for _ in range(2 * NB)]
    + [pltpu.SemaphoreType.DMA for _ in range(2 * NB)]
)


@functools.partial(
    pl.kernel,
    out_type=[
        jax.ShapeDtypeStruct((N,), jnp.float32),
        jax.ShapeDtypeStruct((N,), jnp.float32),
    ],
    mesh=_mesh,
    scratch_types=_scratch,
)
def _sc_field(z_hbm, dx_hbm, dy_hbm, *bufs):
    zin = bufs[0:NB]
    dxo = bufs[NB:2 * NB]
    dyo = bufs[2 * NB:3 * NB]
    in_sem = bufs[3 * NB:4 * NB]
    out_sem = bufs[4 * NB:5 * NB]

    wid = lax.axis_index("s") * NC + lax.axis_index("c")
    base = wid * PER_W

    neg1 = jnp.full((LANES,), -1.0, jnp.float32)
    pos1 = jnp.full((LANES,), 1.0, jnp.float32)

    def start_in(b, ch):
        off = base + ch * CHUNK
        pltpu.async_copy(z_hbm.at[:, pl.ds(off, CHUNK)], zin[b], in_sem[b])

    def wait_in(b):
        pltpu.make_async_copy(z_hbm.at[:, pl.ds(0, CHUNK)], zin[b], in_sem[b]).wait()

    def start_out(b, ch):
        off = base + ch * CHUNK
        pltpu.async_copy(dxo[b], dx_hbm.at[pl.ds(off, CHUNK)], out_sem[b])
        pltpu.async_copy(dyo[b], dy_hbm.at[pl.ds(off, CHUNK)], out_sem[b])

    def wait_out(b):
        pltpu.make_async_copy(dxo[b], dx_hbm.at[pl.ds(0, CHUNK)], out_sem[b]).wait()
        pltpu.make_async_copy(dyo[b], dy_hbm.at[pl.ds(0, CHUNK)], out_sem[b]).wait()

    def compute(b):
        zv, dxv, dyv = zin[b], dxo[b], dyo[b]

        @plsc.parallel_loop(0, CHUNK, step=LANES, unroll=8)
        def _(i):
            s = pl.ds(i, LANES)
            x = zv[0, s]
            y = zv[1, s]
            hot = x >= 2.0
            dxv[s] = jnp.where(hot, -y, jnp.where(y >= 0.0, neg1, pos1))
            dyv[s] = jnp.where(hot, x + 2.0, neg1)

    for b in range(NB):
        start_in(b, jnp.int32(b))

    def group(g, _):
        for b in range(NB):
            ch = g * NB + b
            wait_in(b)

            @pl.when(g > 0)
            def _():
                wait_out(b)

            compute(b)
            start_out(b, ch)

            @pl.when(ch + NB < NCH)
            def _():
                start_in(b, ch + NB)
        return 0

    lax.fori_loop(0, NCH // NB, group, 0)
    for b in range(NB):
        wait_out(b)


def kernel(t, z):
    dx, dy = _sc_field(z)
    return (dx, dy)
